# baseline (speedup 1.0000x reference)
import functools

import jax
import jax.numpy as jnp
from jax import lax
from jax.experimental import pallas as pl
from jax.experimental.pallas import tpu as pltpu
from jax.experimental.pallas import tpu_sc as plsc

NINP = 64
WIDE = 128
PADW = 145
NC = 2
NS = 16
NW = NC * NS
BB = 128


def _body(T, table_hbm, idxT_hbm, outT_hbm, idx_all, in0, in1, ot0, ot1,
          gs0, gs1, os0, os1):
    cid = lax.axis_index("c")
    sid = lax.axis_index("s")
    wid = sid * NC + cid
    bcol = wid * BB

    ins = (in0, in1)
    ots = (ot0, ot1)
    gss = (gs0, gs1)
    oss = (os0, os1)

    pltpu.sync_copy(idxT_hbm.at[:, pl.ds(bcol, BB)], idx_all)

    def fire_gather(t, b):
        pltpu.async_copy(
            table_hbm.at[idx_all.at[t]], ins[b].at[:, pl.ds(0, WIDE)], gss[b]
        )

    def wait_gather(b):
        pltpu.make_async_copy(
            table_hbm.at[pl.ds(0, BB)], ins[b].at[:, pl.ds(0, WIDE)], gss[b]
        ).wait()

    def fire_store(t, b):
        pltpu.async_copy(ots[b], outT_hbm.at[t, :, pl.ds(bcol, BB)], oss[b])

    def drain_store(b):
        pltpu.make_async_copy(
            ots[b], outT_hbm.at[0, :, pl.ds(bcol, BB)], oss[b]
        ).wait()

    rows_all = [lax.iota(jnp.int32, 16) + 16 * m for m in range(BB // 16)]
    JU = 4

    def transpose(b):
        inb = ins[b]
        otb = ots[b]

        def jbody(j4, carry):
            j0 = j4 * JU
            vs = []
            for dj in range(JU):
                cols = jnp.full((16,), j0 + dj, jnp.int32)
                for m in range(BB // 16):
                    vs.append(plsc.load_gather(inb, [rows_all[m], cols]))
            i = 0
            for dj in range(JU):
                for m in range(BB // 16):
                    otb[j0 + dj, pl.ds(16 * m, 16)] = vs[i]
                    i += 1
            return carry

        lax.fori_loop(0, NINP // JU, jbody, 0)

    def visit(t, b, first, fire_next):
        if not first:
            drain_store(b)
        wait_gather(b)
        if fire_next:
            fire_gather(t + 1, 1 - b)
        if transpose is not None:
            transpose(b)
        fire_store(t, b)

    fire_gather(0, 0)
    visit(0, 0, True, True)
    visit(1, 1, True, True)

    def steady(i, carry):
        t = 2 * i
        visit(t, 0, False, True)
        visit(t + 1, 1, False, True)
        return carry

    lax.fori_loop(1, T // 2 - 1, steady, 0)

    visit(T - 2, 0, False, True)
    visit(T - 1, 1, False, False)
    drain_store(0)
    drain_store(1)


def kernel(input, word_embed_weight):
    B, T = input.shape
    V = word_embed_weight.shape[0]
    assert B == NW * BB and T % 2 == 0

    idxT = jnp.transpose(input).astype(jnp.int32)
    pad = jnp.zeros((V, WIDE - NINP), jnp.float32)
    wide = jnp.concatenate([word_embed_weight, pad], axis=1)

    mesh = plsc.VectorSubcoreMesh(core_axis_name="c", subcore_axis_name="s")
    k = functools.partial(
        pl.kernel,
        mesh=mesh,
        out_type=jax.ShapeDtypeStruct((T, NINP, B), jnp.float32),
        scratch_types=[
            pltpu.VMEM((T, BB), jnp.int32),
            pltpu.VMEM((BB, PADW), jnp.float32),
            pltpu.VMEM((BB, PADW), jnp.float32),
            pltpu.VMEM((NINP, BB), jnp.float32),
            pltpu.VMEM((NINP, BB), jnp.float32),
            pltpu.SemaphoreType.DMA,
            pltpu.SemaphoreType.DMA,
            pltpu.SemaphoreType.DMA,
            pltpu.SemaphoreType.DMA,
        ],
        compiler_params=pltpu.CompilerParams(needs_layout_passes=False),
    )(functools.partial(_body, T))

    outT = k(wide, idxT)
    return jnp.transpose(outT, (2, 0, 1))


# speedup vs baseline: 1.2370x; 1.2370x over previous
import functools

import jax
import jax.numpy as jnp
from jax import lax
from jax.experimental import pallas as pl
from jax.experimental.pallas import tpu as pltpu
from jax.experimental.pallas import tpu_sc as plsc

NINP = 64
NC = 2
NS = 16
NW = NC * NS
G = 128
S = 2
R = G * S
NBUF = 4


def _gather_body(n_chunks, table_hbm, idx_hbm, out_hbm, idx_v, rows_v, gsems, osems):
    cid = lax.axis_index("c")
    sid = lax.axis_index("s")
    wid = sid * NC + cid
    crow0 = wid * (n_chunks * S)
    row0 = wid * (n_chunks * R)

    pltpu.sync_copy(idx_hbm.at[pl.ds(crow0, n_chunks * S)], idx_v)

    def fire_gathers(g, b):
        for j in range(S):
            pltpu.async_copy(
                table_hbm.at[idx_v.at[g * S + j]],
                rows_v.at[b].at[pl.ds(j * G, G)],
                gsems[b],
            )

    def drain_gathers(b):
        for j in range(S):
            pltpu.make_async_copy(
                table_hbm.at[pl.ds(0, G)],
                rows_v.at[b].at[pl.ds(j * G, G)],
                gsems[b],
            ).wait()

    def fire_store(g, b):
        pltpu.async_copy(
            rows_v.at[b], out_hbm.at[pl.ds(row0 + g * R, R)], osems[b]
        )

    def drain_store(b):
        pltpu.make_async_copy(
            rows_v.at[b], out_hbm.at[pl.ds(row0, R)], osems[b]
        ).wait()

    n = n_chunks
    fire_gathers(0, 0)
    for b in range(NBUF - 1):
        fire_gathers(b + 1, b + 1)
        drain_gathers(b)
        fire_store(b, b)
    b = NBUF - 1
    drain_store(0)
    fire_gathers(NBUF, 0)
    drain_gathers(b)
    fire_store(b, b)

    def rotation(i, carry):
        for b in range(NBUF):
            g = i * NBUF + b
            b1 = (b + 1) % NBUF
            drain_store(b1)
            fire_gathers(g + 1, b1)
            drain_gathers(b)
            fire_store(g, b)
        return carry

    lax.fori_loop(1, n // NBUF - 1, rotation, 0)

    for b in range(NBUF):
        g = n - NBUF + b
        if b < NBUF - 1:
            b1 = (b + 1) % NBUF
            drain_store(b1)
            fire_gathers(g + 1, b1)
        drain_gathers(b)
        fire_store(g, b)
    for b in range(NBUF):
        drain_store(b)


def kernel(input, word_embed_weight):
    B = input.shape[0] * input.shape[1]
    assert B % (NW * R) == 0
    n_chunks = B // (NW * R)

    idx2 = input.reshape(B // G, G).astype(jnp.int32)

    mesh = plsc.VectorSubcoreMesh(core_axis_name="c", subcore_axis_name="s")
    k = functools.partial(
        pl.kernel,
        mesh=mesh,
        out_type=jax.ShapeDtypeStruct((B, NINP), jnp.float32),
        scratch_types=[
            pltpu.VMEM((n_chunks * S, G), jnp.int32),
            pltpu.VMEM((NBUF, R, NINP), jnp.float32),
            [pltpu.SemaphoreType.DMA] * NBUF,
            [pltpu.SemaphoreType.DMA] * NBUF,
        ],
        compiler_params=pltpu.CompilerParams(use_tc_tiling_on_sc=False),
    )(functools.partial(_gather_body, n_chunks))

    out = k(word_embed_weight, idx2)
    return out.reshape(input.shape[0], input.shape[1], NINP)
